# baseline (speedup 1.0000x reference)
import functools

import jax
import jax.numpy as jnp
from jax import lax
from jax.experimental import pallas as pl
from jax.experimental.pallas import tpu as pltpu
from jax.experimental.pallas import tpu_sc as plsc

DIM = 128
NSEQ = 4096
SEQLEN = 50
SEQPAD = 128
NSPLIT = 2

_info = plsc.get_sparse_core_info()
_NC, _NS = _info.num_cores, _info.num_subcores
_NW = _NC * _NS
_SCHUNK = 8


def _make_kernel(nseq):
  spw = nseq // _NW
  nchunk = spw // _SCHUNK
  nbuf = 2
  mesh = plsc.VectorSubcoreMesh(core_axis_name="c", subcore_axis_name="s")

  @functools.partial(
      pl.kernel,
      mesh=mesh,
      out_type=jax.ShapeDtypeStruct((nseq, SEQLEN, DIM), jnp.float32),
      scratch_types=(
          [pltpu.VMEM((spw, SEQPAD), jnp.int32)]
          + [pltpu.VMEM((_SCHUNK, SEQLEN, DIM), jnp.float32)
             for _ in range(nbuf)]
          + [pltpu.SemaphoreType.DMA for _ in range(2 * nbuf)]
      ),
  )
  def emb_kernel(idx_hbm, table_hbm, out_hbm, idx_v, *scratch):
    rows = scratch[:nbuf]
    gsem = scratch[nbuf:2 * nbuf]
    ssem = scratch[2 * nbuf:]
    wid = lax.axis_index("s") * _NC + lax.axis_index("c")
    seq0 = wid * spw
    pltpu.sync_copy(idx_hbm.at[pl.ds(seq0, spw)], idx_v)

    def fire_gathers(c, b):
      for s in range(_SCHUNK):
        idx_ref = idx_v.at[c * _SCHUNK + s, pl.ds(0, SEQLEN)]
        pltpu.async_copy(table_hbm.at[idx_ref], rows[b].at[s], gsem[b])

    def fire_stores(c, b):
      for s in range(_SCHUNK):
        pltpu.async_copy(rows[b].at[s], out_hbm.at[seq0 + c * _SCHUNK + s],
                         ssem[b])

    def drain_gathers(b):
      for s in range(_SCHUNK):
        pltpu.make_async_copy(table_hbm.at[idx_v.at[0, pl.ds(0, SEQLEN)]],
                              rows[b].at[s], gsem[b]).wait()

    def drain_stores(c, b):
      for s in range(_SCHUNK):
        pltpu.make_async_copy(rows[b].at[s],
                              out_hbm.at[seq0 + c * _SCHUNK + s],
                              ssem[b]).wait()

    fire_gathers(0, 0)

    def body(i, carry):
      for b in range(nbuf):
        c = nbuf * i + b
        drain_gathers(b)
        fire_stores(c, b)
        nb = (b + 1) % nbuf

        @pl.when(c + 1 < nchunk)
        def _():
          @pl.when(c >= 1)
          def _():
            drain_stores(c - 1, nb)
          fire_gathers(c + 1, nb)
      return carry

    lax.fori_loop(0, nchunk // nbuf, body, 0)

    for b in range(nbuf):
      drain_stores(nchunk - nbuf + b, b)

  return emb_kernel


_emb = _make_kernel(NSEQ // NSPLIT)


@jax.jit
def kernel(token_ids, weight):
  idx = jnp.pad(token_ids.astype(jnp.int32),
                ((0, 0), (0, SEQPAD - SEQLEN)))
  h = NSEQ // NSPLIT
  parts = [_emb(idx[k * h:(k + 1) * h], weight) for k in range(NSPLIT)]
  return jnp.concatenate(parts, axis=0)


# speedup vs baseline: 1.6234x; 1.6234x over previous
import functools

import jax
from jax.experimental import layout as jlayout
import jax.numpy as jnp
from jax import lax
from jax.experimental import pallas as pl
from jax.experimental.pallas import tpu as pltpu
from jax.experimental.pallas import tpu_sc as plsc

DIM = 128
NSEQ = 4096
SEQLEN = 50
SEQPAD = 128

_info = plsc.get_sparse_core_info()
_NC, _NS = _info.num_cores, _info.num_subcores
_NW = _NC * _NS
_SPW = NSEQ // _NW
_SCHUNK = 8
_NCHUNK = _SPW // _SCHUNK
_NBUF = 2


def _make_kernel():
  mesh = plsc.VectorSubcoreMesh(core_axis_name="c", subcore_axis_name="s")

  @functools.partial(
      pl.kernel,
      mesh=mesh,
      out_type=jax.ShapeDtypeStruct((NSEQ, SEQLEN, DIM), jnp.float32),
      compiler_params=pltpu.CompilerParams(use_tc_tiling_on_sc=True),
      scratch_types=(
          [pltpu.VMEM((_SPW, SEQPAD), jnp.int32)]
          + [pltpu.VMEM((_SCHUNK, SEQLEN, DIM), jnp.float32)
             for _ in range(_NBUF)]
          + [pltpu.SemaphoreType.DMA for _ in range(2 * _NBUF)]
      ),
  )
  def emb_kernel(idx_hbm, table_hbm, out_hbm, idx_v, *scratch):
    rows = scratch[:_NBUF]
    gsem = scratch[_NBUF:2 * _NBUF]
    ssem = scratch[2 * _NBUF:]
    wid = lax.axis_index("s") * _NC + lax.axis_index("c")
    seq0 = wid * _SPW
    pltpu.sync_copy(idx_hbm.at[pl.ds(seq0, _SPW)], idx_v)

    def fire_gathers(c, b):
      for s in range(_SCHUNK):
        idx_ref = idx_v.at[c * _SCHUNK + s, pl.ds(0, SEQLEN)]
        pltpu.async_copy(table_hbm.at[idx_ref], rows[b].at[s], gsem[b])

    def fire_stores(c, b):
      for s in range(_SCHUNK):
        pltpu.async_copy(rows[b].at[s], out_hbm.at[seq0 + c * _SCHUNK + s],
                         ssem[b])

    def drain_gathers(b):
      for s in range(_SCHUNK):
        pltpu.make_async_copy(table_hbm.at[idx_v.at[0, pl.ds(0, SEQLEN)]],
                              rows[b].at[s], gsem[b]).wait()

    def drain_stores(c, b):
      for s in range(_SCHUNK):
        pltpu.make_async_copy(rows[b].at[s],
                              out_hbm.at[seq0 + c * _SCHUNK + s],
                              ssem[b]).wait()

    fire_gathers(0, 0)

    def body(i, carry):
      for b in range(_NBUF):
        c = _NBUF * i + b
        drain_gathers(b)
        fire_stores(c, b)
        nb = (b + 1) % _NBUF

        @pl.when(c + 1 < _NCHUNK)
        def _():
          @pl.when(c >= 1)
          def _():
            drain_stores(c - 1, nb)
          fire_gathers(c + 1, nb)
      return carry

    lax.fori_loop(0, _NCHUNK // _NBUF, body, 0)

    for b in range(_NBUF):
      drain_stores(_NCHUNK - _NBUF + b, b)

  return emb_kernel


_emb = _make_kernel()


def _impl(token_ids, weight):
  idx = jnp.pad(token_ids.astype(jnp.int32),
                ((0, 0), (0, SEQPAD - SEQLEN)))
  return _emb(idx, weight)


def _make_jit():
  fmt = jlayout.Format(
      jlayout.Layout(major_to_minor=(0, 1, 2), tiling=()),
      jax.sharding.SingleDeviceSharding(jax.devices()[0]))
  return jax.jit(_impl, out_shardings=fmt)


_jitted = None


def kernel(token_ids, weight):
  global _jitted
  if _jitted is None:
    _jitted = _make_jit()
  return _jitted(token_ids, weight)


# speedup vs baseline: 1.6508x; 1.0169x over previous
import functools

import jax
import jax.numpy as jnp
from jax import lax
from jax.experimental import pallas as pl
from jax.experimental.pallas import tpu as pltpu
from jax.experimental.pallas import tpu_sc as plsc

DIM = 128
NSEQ = 4096
SEQLEN = 50
SEQPAD = 128

_info = plsc.get_sparse_core_info()
_NC, _NS = _info.num_cores, _info.num_subcores
_NW = _NC * _NS
_SPW = NSEQ // _NW
_SCHUNK = 4
_NCHUNK = _SPW // _SCHUNK
_NBUF = 4


def _make_kernel():
  mesh = plsc.VectorSubcoreMesh(core_axis_name="c", subcore_axis_name="s")

  @functools.partial(
      pl.kernel,
      mesh=mesh,
      out_type=jax.ShapeDtypeStruct((NSEQ, SEQLEN, DIM), jnp.float32),
      compiler_params=pltpu.CompilerParams(use_tc_tiling_on_sc=True),
      scratch_types=(
          [pltpu.VMEM((_SPW, SEQPAD), jnp.int32)]
          + [pltpu.VMEM((_SCHUNK, SEQLEN, DIM), jnp.float32)
             for _ in range(_NBUF)]
          + [pltpu.SemaphoreType.DMA for _ in range(2 * _NBUF)]
      ),
  )
  def emb_kernel(idx_hbm, table_hbm, out_hbm, idx_v, *scratch):
    rows = scratch[:_NBUF]
    gsem = scratch[_NBUF:2 * _NBUF]
    ssem = scratch[2 * _NBUF:]
    wid = lax.axis_index("s") * _NC + lax.axis_index("c")
    seq0 = wid * _SPW
    pltpu.sync_copy(idx_hbm.at[pl.ds(seq0, _SPW)], idx_v)

    def fire_gathers(c, b):
      for s in range(_SCHUNK):
        idx_ref = idx_v.at[c * _SCHUNK + s, pl.ds(0, SEQLEN)]
        pltpu.async_copy(table_hbm.at[idx_ref], rows[b].at[s], gsem[b])

    def fire_stores(c, b):
      for s in range(_SCHUNK):
        pltpu.async_copy(rows[b].at[s], out_hbm.at[seq0 + c * _SCHUNK + s],
                         ssem[b])

    def drain_gathers(b):
      for s in range(_SCHUNK):
        pltpu.make_async_copy(table_hbm.at[idx_v.at[0, pl.ds(0, SEQLEN)]],
                              rows[b].at[s], gsem[b]).wait()

    def drain_stores(c, b):
      for s in range(_SCHUNK):
        pltpu.make_async_copy(rows[b].at[s],
                              out_hbm.at[seq0 + c * _SCHUNK + s],
                              ssem[b]).wait()

    for b in range(_NBUF - 1):
      fire_gathers(b, b)

    def body(i, carry):
      for b in range(_NBUF):
        c = _NBUF * i + b
        drain_gathers(b)
        fire_stores(c, b)
        nb = (b + _NBUF - 1) % _NBUF

        @pl.when(c + _NBUF - 1 < _NCHUNK)
        def _():
          @pl.when(c >= 1)
          def _():
            drain_stores(c - 1, nb)
          fire_gathers(c + _NBUF - 1, nb)
      return carry

    lax.fori_loop(0, _NCHUNK // _NBUF, body, 0)

    for b in range(_NBUF):
      drain_stores(_NCHUNK - _NBUF + b, b)

  return emb_kernel


_emb = _make_kernel()


def _impl(token_ids, weight):
  idx = jnp.pad(token_ids.astype(jnp.int32),
                ((0, 0), (0, SEQPAD - SEQLEN)))
  return _emb(idx, weight)


kernel = jax.jit(_impl)
